# baseline (speedup 1.0000x reference)
import jax
import jax.numpy as jnp
from jax.experimental import pallas as pl
from jax.experimental.pallas import tpu as pltpu

P = 50
B = 128
N = 784
DEG = 4
LL = 2
MAX_ITER_SEG = 3
NUM_COLORS = 2
KTOT = MAX_ITER_SEG + LL
DP = float(DEG)

BCG = 16
BCR = 16

_PREC = jax.lax.Precision.HIGHEST


def _gram_kernel(x_ref, a0_ref, ga_ref, gx_ref):
    x = x_ref[...]
    a0 = a0_ref[...]
    dn = (((2,), (2,)), ((1,), (1,)))
    ga_ref[...] = jax.lax.dot_general(x, a0, dn, precision=_PREC,
                                      preferred_element_type=jnp.float32)
    gx_ref[...] = jax.lax.dot_general(x, x, dn, precision=_PREC,
                                      preferred_element_type=jnp.float32)


def _iter_kernel(nbr_ref, cid_ref, hs_ref, gat_ref, gxt_ref, om0_ref, lab_ref,
                 u_ref, v_ref, om_ref, m_s, r_s, su_s, sv_s, som_s):
    iota_p = jax.lax.broadcasted_iota(jnp.int32, (P, P), 0)
    iota_q = jax.lax.broadcasted_iota(jnp.int32, (P, P), 1)
    eye = (iota_p == iota_q).astype(jnp.float32)[:, :, None]
    u_ref[...] = jnp.broadcast_to(eye, (P, P, B))
    v_ref[...] = jnp.zeros((P, P, B), jnp.float32)
    m_s[...] = jnp.zeros((P, P, B), jnp.float32)
    r_s[...] = jnp.zeros((P, P, B), jnp.float32)
    om_ref[...] = om0_ref[...]
    gat = gat_ref[...]
    gxt = gxt_ref[...]
    lab = lab_ref[...]
    cid = cid_ref[...]
    lam = jnp.zeros((P, 1, B), jnp.float32)

    def nsum_body(p, carry):
        n0 = nbr_ref[p, 0]
        n1 = nbr_ref[p, 1]
        n2 = nbr_ref[p, 2]
        n3 = nbr_ref[p, 3]
        su_s[p] = u_ref[n0] + u_ref[n1] + u_ref[n2] + u_ref[n3]
        sv_s[p] = v_ref[n0] + v_ref[n1] + v_ref[n2] + v_ref[n3]
        som_s[p] = om_ref[n0] + om_ref[n1] + om_ref[n2] + om_ref[n3]
        return carry

    def nsum():
        jax.lax.fori_loop(0, P, nsum_body, 0)

    nsum()
    for k in range(KTOT):
        h = jnp.abs(hs_ref[k])
        h0 = h[:, 0:1][:, :, None]
        h1 = h[:, 1:2][:, :, None]
        h2 = h[:, 2:3][:, :, None]
        h3 = h[:, 3:4][:, :, None]
        h4 = h[:, 4:5][:, :, None]
        h5 = h[:, 5:6][:, :, None]
        for color in range(NUM_COLORS):
            u = u_ref[...]
            v = v_ref[...]
            om = om_ref[...]
            s = jnp.sum(u * gat + v * gxt, axis=1, keepdims=True)
            c = s + om - lab
            u_new = u - h1 * ((h0 * DP) * u + DP * m_s[...] - h0 * su_s[...])
            v_new = (v - h1 * ((h0 * DP) * v + DP * r_s[...] - h0 * sv_s[...])
                     - (h1 * c) * eye)
            om_new = om - h5 * (c + (h2 * DP) * om + DP * lam - h2 * som_s[...])
            mask = (cid == color)[:, :, None]
            u_ref[...] = jnp.where(mask, u_new, u)
            v_ref[...] = jnp.where(mask, v_new, v)
            om_ref[...] = jnp.where(mask, om_new, om)
            nsum()
        m_s[...] = m_s[...] + h3 * (DP * u_ref[...] - su_s[...])
        r_s[...] = r_s[...] + h3 * (DP * v_ref[...] - sv_s[...])
        lam = lam + h4 * (DP * om_ref[...] - som_s[...])


def _recon_kernel(u_ref, v_ref, a0_ref, x_ref, out_ref):
    dn = (((2,), (0,)), ((0,), (1,)))
    out_ref[...] = (
        jax.lax.dot_general(u_ref[...], a0_ref[...], dn, precision=_PREC,
                            preferred_element_type=jnp.float32)
        + jax.lax.dot_general(v_ref[...], x_ref[...], dn, precision=_PREC,
                              preferred_element_type=jnp.float32))


def kernel(inputs, labels, hyp, no_hyp, neighbors, color_ids):
    x = inputs.reshape(P, B, N)
    lab = labels.reshape(P, B)[:, None, :]
    hs = jnp.concatenate([no_hyp, hyp], axis=0)
    nbr = neighbors.astype(jnp.int32)
    cid = color_ids.astype(jnp.int32).reshape(P, 1)

    kinit = jax.random.key(1234)
    ka, ko = jax.random.split(kinit)
    a0 = jax.random.normal(ka, (P, B, N, 1), dtype=jnp.float32).reshape(P, B, N)
    om0 = jax.random.uniform(ko, (P, B, 1, 1), dtype=jnp.float32)
    om0 = om0.reshape(P, B)[:, None, :]

    big = lambda i: (0, i, 0)
    ga, gx = pl.pallas_call(
        _gram_kernel,
        grid=(B // BCG,),
        in_specs=[
            pl.BlockSpec((P, BCG, N), big),
            pl.BlockSpec((P, BCG, N), big),
        ],
        out_specs=[
            pl.BlockSpec((BCG, P, P), lambda i: (i, 0, 0)),
            pl.BlockSpec((BCG, P, P), lambda i: (i, 0, 0)),
        ],
        out_shape=[
            jax.ShapeDtypeStruct((B, P, P), jnp.float32),
            jax.ShapeDtypeStruct((B, P, P), jnp.float32),
        ],
    )(x, a0)
    gat = jnp.transpose(ga, (1, 2, 0))
    gxt = jnp.transpose(gx, (1, 2, 0))

    full3 = lambda shape: pl.BlockSpec(shape, lambda i: (0,) * len(shape))
    uo, vo, omo = pl.pallas_call(
        _iter_kernel,
        grid=(1,),
        in_specs=[
            pl.BlockSpec(memory_space=pltpu.SMEM),
            full3((P, 1)),
            full3((KTOT, P, 6)),
            full3((P, P, B)),
            full3((P, P, B)),
            full3((P, 1, B)),
            full3((P, 1, B)),
        ],
        out_specs=[
            full3((P, P, B)),
            full3((P, P, B)),
            full3((P, 1, B)),
        ],
        out_shape=[
            jax.ShapeDtypeStruct((P, P, B), jnp.float32),
            jax.ShapeDtypeStruct((P, P, B), jnp.float32),
            jax.ShapeDtypeStruct((P, 1, B), jnp.float32),
        ],
        scratch_shapes=[
            pltpu.VMEM((P, P, B), jnp.float32),
            pltpu.VMEM((P, P, B), jnp.float32),
            pltpu.VMEM((P, P, B), jnp.float32),
            pltpu.VMEM((P, P, B), jnp.float32),
            pltpu.VMEM((P, 1, B), jnp.float32),
        ],
    )(nbr, cid, hs, gat, gxt, om0, lab)

    ut = jnp.transpose(uo, (2, 0, 1))
    vt = jnp.transpose(vo, (2, 0, 1))
    a_bpn = pl.pallas_call(
        _recon_kernel,
        grid=(B // BCR,),
        in_specs=[
            pl.BlockSpec((BCR, P, P), lambda i: (i, 0, 0)),
            pl.BlockSpec((BCR, P, P), lambda i: (i, 0, 0)),
            pl.BlockSpec((P, BCR, N), big),
            pl.BlockSpec((P, BCR, N), big),
        ],
        out_specs=pl.BlockSpec((BCR, P, N), lambda i: (i, 0, 0)),
        out_shape=jax.ShapeDtypeStruct((B, P, N), jnp.float32),
    )(ut, vt, a0, x)

    a_out = jnp.swapaxes(a_bpn, 0, 1).reshape(P, B, N, 1)
    om_out = jnp.transpose(omo, (0, 2, 1)).reshape(P, B, 1, 1)
    return a_out, om_out


# speedup vs baseline: 1.3171x; 1.3171x over previous
import jax
import jax.numpy as jnp
from jax.experimental import pallas as pl
from jax.experimental.pallas import tpu as pltpu

P = 50
B = 128
N = 784
DEG = 4
LL = 2
MAX_ITER_SEG = 3
NUM_COLORS = 2
KTOT = MAX_ITER_SEG + LL
DP = float(DEG)
Q2 = 2 * P

BCG = 32
BCR = 32


def _dot3(lhs, rhs, dn):
    lh = lhs.astype(jnp.bfloat16)
    ll = (lhs - lh.astype(jnp.float32)).astype(jnp.bfloat16)
    rh = rhs.astype(jnp.bfloat16)
    rl = (rhs - rh.astype(jnp.float32)).astype(jnp.bfloat16)
    f32 = jnp.float32
    return (jax.lax.dot_general(lh, rh, dn, preferred_element_type=f32)
            + jax.lax.dot_general(lh, rl, dn, preferred_element_type=f32)
            + jax.lax.dot_general(ll, rh, dn, preferred_element_type=f32))


def _gram_kernel(x_ref, a0_ref, g_ref):
    x = x_ref[...]
    a0 = a0_ref[...]
    cat = jnp.concatenate([a0, x], axis=0)
    dn = (((2,), (2,)), ((1,), (1,)))
    g_ref[...] = _dot3(x, cat, dn)


def _iter_kernel(nbr_ref, cid_ref, hs_ref, g_ref, om0_ref, lab_ref,
                 w_ref, om_ref, m_s, sw_s, som_s):
    ip = jax.lax.broadcasted_iota(jnp.int32, (P, Q2), 0)
    iq = jax.lax.broadcasted_iota(jnp.int32, (P, Q2), 1)
    eye_u = (iq == ip).astype(jnp.float32)[:, :, None]
    eye_v = (iq == ip + P).astype(jnp.float32)[:, :, None]
    w_ref[...] = jnp.broadcast_to(eye_u, (P, Q2, B))
    m_s[...] = jnp.zeros((P, Q2, B), jnp.float32)
    om_ref[...] = om0_ref[...]
    g = g_ref[...]
    lab = lab_ref[...]
    cid = cid_ref[...]
    lam = jnp.zeros((P, 1, B), jnp.float32)

    def nsum_body(p, carry):
        n0 = nbr_ref[p, 0]
        n1 = nbr_ref[p, 1]
        n2 = nbr_ref[p, 2]
        n3 = nbr_ref[p, 3]
        sw_s[p] = w_ref[n0] + w_ref[n1] + w_ref[n2] + w_ref[n3]
        som_s[p] = om_ref[n0] + om_ref[n1] + om_ref[n2] + om_ref[n3]
        return carry

    def nsum():
        jax.lax.fori_loop(0, P, nsum_body, 0)

    nsum()
    for k in range(KTOT):
        h = jnp.abs(hs_ref[k])
        h0 = h[:, 0:1][:, :, None]
        h1 = h[:, 1:2][:, :, None]
        h2 = h[:, 2:3][:, :, None]
        h3 = h[:, 3:4][:, :, None]
        h4 = h[:, 4:5][:, :, None]
        h5 = h[:, 5:6][:, :, None]
        for color in range(NUM_COLORS):
            w = w_ref[...]
            om = om_ref[...]
            s = jnp.sum(w * g, axis=1, keepdims=True)
            c = s + om - lab
            w_new = (w - h1 * ((h0 * DP) * w + DP * m_s[...] - h0 * sw_s[...])
                     - (h1 * c) * eye_v)
            om_new = om - h5 * (c + (h2 * DP) * om + DP * lam - h2 * som_s[...])
            mask = (cid == color)[:, :, None]
            w_ref[...] = jnp.where(mask, w_new, w)
            om_ref[...] = jnp.where(mask, om_new, om)
            nsum()
        m_s[...] = m_s[...] + h3 * (DP * w_ref[...] - sw_s[...])
        lam = lam + h4 * (DP * om_ref[...] - som_s[...])
    w_ref[...] = w_ref[...] - jnp.broadcast_to(eye_u, (P, Q2, B))


def _recon_kernel(w_ref, a0_ref, x_ref, out_ref):
    w = w_ref[...]
    a0 = a0_ref[...]
    cat = jnp.concatenate([a0, x_ref[...]], axis=0)
    dn = (((2,), (0,)), ((0,), (1,)))
    d = _dot3(w, cat, dn)
    out_ref[...] = jnp.swapaxes(d, 0, 1) + a0


def kernel(inputs, labels, hyp, no_hyp, neighbors, color_ids):
    x = inputs.reshape(P, B, N)
    lab = labels.reshape(P, B)[:, None, :]
    hs = jnp.concatenate([no_hyp, hyp], axis=0)
    nbr = neighbors.astype(jnp.int32)
    cid = color_ids.astype(jnp.int32).reshape(P, 1)

    kinit = jax.random.key(1234)
    ka, ko = jax.random.split(kinit)
    a0 = jax.random.normal(ka, (P, B, N, 1), dtype=jnp.float32).reshape(P, B, N)
    om0 = jax.random.uniform(ko, (P, B, 1, 1), dtype=jnp.float32)
    om0 = om0.reshape(P, B)[:, None, :]

    big = lambda i: (0, i, 0)
    gbpq = pl.pallas_call(
        _gram_kernel,
        grid=(B // BCG,),
        in_specs=[
            pl.BlockSpec((P, BCG, N), big),
            pl.BlockSpec((P, BCG, N), big),
        ],
        out_specs=pl.BlockSpec((BCG, P, Q2), lambda i: (i, 0, 0)),
        out_shape=jax.ShapeDtypeStruct((B, P, Q2), jnp.float32),
    )(x, a0)
    g = jnp.transpose(gbpq, (1, 2, 0))

    full3 = lambda shape: pl.BlockSpec(shape, lambda i: (0,) * len(shape))
    wo, omo = pl.pallas_call(
        _iter_kernel,
        grid=(1,),
        in_specs=[
            pl.BlockSpec(memory_space=pltpu.SMEM),
            full3((P, 1)),
            full3((KTOT, P, 6)),
            full3((P, Q2, B)),
            full3((P, 1, B)),
            full3((P, 1, B)),
        ],
        out_specs=[
            full3((P, Q2, B)),
            full3((P, 1, B)),
        ],
        out_shape=[
            jax.ShapeDtypeStruct((P, Q2, B), jnp.float32),
            jax.ShapeDtypeStruct((P, 1, B), jnp.float32),
        ],
        scratch_shapes=[
            pltpu.VMEM((P, Q2, B), jnp.float32),
            pltpu.VMEM((P, Q2, B), jnp.float32),
            pltpu.VMEM((P, 1, B), jnp.float32),
        ],
    )(nbr, cid, hs, g, om0, lab)

    wt = jnp.transpose(wo, (2, 0, 1))
    a_pbn = pl.pallas_call(
        _recon_kernel,
        grid=(B // BCR,),
        in_specs=[
            pl.BlockSpec((BCR, P, Q2), lambda i: (i, 0, 0)),
            pl.BlockSpec((P, BCR, N), big),
            pl.BlockSpec((P, BCR, N), big),
        ],
        out_specs=pl.BlockSpec((P, BCR, N), big),
        out_shape=jax.ShapeDtypeStruct((P, B, N), jnp.float32),
    )(wt, a0, x)

    a_out = a_pbn.reshape(P, B, N, 1)
    om_out = jnp.transpose(omo, (0, 2, 1)).reshape(P, B, 1, 1)
    return a_out, om_out


# speedup vs baseline: 1.8391x; 1.3964x over previous
import jax
import jax.numpy as jnp
from jax.experimental import pallas as pl
from jax.experimental.pallas import tpu as pltpu

P = 50
B = 128
N = 784
DEG = 4
LL = 2
MAX_ITER_SEG = 3
NUM_COLORS = 2
KTOT = MAX_ITER_SEG + LL
DP = float(DEG)
Q2 = 2 * P
QB = Q2 * B

BCG = 32
BCR = 32


def _dot3(lhs, rhs, dn):
    lh = lhs.astype(jnp.bfloat16)
    ll = (lhs - lh.astype(jnp.float32)).astype(jnp.bfloat16)
    rh = rhs.astype(jnp.bfloat16)
    rl = (rhs - rh.astype(jnp.float32)).astype(jnp.bfloat16)
    f32 = jnp.float32
    return (jax.lax.dot_general(lh, rh, dn, preferred_element_type=f32)
            + jax.lax.dot_general(lh, rl, dn, preferred_element_type=f32)
            + jax.lax.dot_general(ll, rh, dn, preferred_element_type=f32))


def _gram_kernel(x_ref, a0_ref, g_ref):
    xt = jnp.swapaxes(x_ref[...], 0, 1)
    at = jnp.swapaxes(a0_ref[...], 0, 1)
    cat = jnp.concatenate([at, xt], axis=1)
    dn = (((2,), (2,)), ((0,), (0,)))
    g_ref[...] = _dot3(xt, cat, dn)


def _iter_kernel(nbr_ref, cid_ref, hs_ref, g_ref, om0_ref, lab_ref,
                 w_ref, om_ref):
    ip = jax.lax.broadcasted_iota(jnp.int32, (P, QB), 0)
    iqb = jax.lax.broadcasted_iota(jnp.int32, (P, QB), 1)
    qidx = jax.lax.shift_right_logical(iqb, 7)
    m_u = qidx == ip
    m_v = qidx == ip + P
    one = jnp.float32(1.0)
    w = jnp.where(m_u, one, 0.0)
    m = jnp.zeros((P, QB), jnp.float32)
    om = om0_ref[...]
    lam = jnp.zeros((P, B), jnp.float32)
    g = g_ref[...]
    lab = lab_ref[...]
    cid = cid_ref[...]

    iota_r = jax.lax.broadcasted_iota(jnp.int32, (P, P), 1)
    adj = jnp.zeros((P, P), jnp.float32)
    for d in range(DEG):
        adj = adj + (iota_r == nbr_ref[:, d:d + 1]).astype(jnp.float32)
    adjb = adj.astype(jnp.bfloat16)

    dn2 = (((1,), (0,)), ((), ()))

    def nsum(wv, omv):
        sw = jax.lax.dot_general(adjb, wv.astype(jnp.bfloat16), dn2,
                                 preferred_element_type=jnp.float32)
        som = jax.lax.dot_general(adjb, omv.astype(jnp.bfloat16), dn2,
                                  preferred_element_type=jnp.float32)
        return sw, som

    def rowsum_q(prod):
        quarter = QB // 4
        x = (prod[:, :quarter] + prod[:, quarter:2 * quarter]
             + prod[:, 2 * quarter:3 * quarter] + prod[:, 3 * quarter:])
        g5 = 5 * B
        x = (x[:, :g5] + x[:, g5:2 * g5] + x[:, 2 * g5:3 * g5]
             + x[:, 3 * g5:4 * g5] + x[:, 4 * g5:])
        return (x[:, :B] + x[:, B:2 * B] + x[:, 2 * B:3 * B]
                + x[:, 3 * B:4 * B] + x[:, 4 * B:])

    sw, som = nsum(w, om)
    for k in range(KTOT):
        h = jnp.abs(hs_ref[k])
        h0 = h[:, 0:1]
        h1 = h[:, 1:2]
        h2 = h[:, 2:3]
        h3 = h[:, 3:4]
        h4 = h[:, 4:5]
        h5 = h[:, 5:6]
        for color in range(NUM_COLORS):
            s = rowsum_q(w * g)
            c = s + om - lab
            w_new = (w - h1 * ((h0 * DP) * w + DP * m - h0 * sw)
                     - jnp.where(m_v, jnp.tile(h1 * c, (1, Q2)), 0.0))
            om_new = om - h5 * (c + (h2 * DP) * om + DP * lam - h2 * som)
            cmask = cid == color
            w = jnp.where(cmask, w_new, w)
            om = jnp.where(cmask, om_new, om)
            sw, som = nsum(w, om)
        m = m + h3 * (DP * w - sw)
        lam = lam + h4 * (DP * om - som)
    w_ref[...] = w - jnp.where(m_u, one, 0.0)
    om_ref[...] = om


def _recon_kernel(w_ref, a0_ref, x_ref, out_ref):
    w = w_ref[...]
    a0 = a0_ref[...]
    at = jnp.swapaxes(a0, 0, 1)
    xt = jnp.swapaxes(x_ref[...], 0, 1)
    cat = jnp.concatenate([at, xt], axis=1)
    dn = (((2,), (1,)), ((0,), (0,)))
    d = _dot3(w, cat, dn)
    out_ref[...] = jnp.swapaxes(d, 0, 1) + a0


def kernel(inputs, labels, hyp, no_hyp, neighbors, color_ids):
    x = inputs.reshape(P, B, N)
    lab = labels.reshape(P, B)
    hs = jnp.concatenate([no_hyp, hyp], axis=0)
    nbr = neighbors.astype(jnp.int32)
    cid = color_ids.astype(jnp.int32).reshape(P, 1)

    kinit = jax.random.key(1234)
    ka, ko = jax.random.split(kinit)
    a0 = jax.random.normal(ka, (P, B, N, 1), dtype=jnp.float32).reshape(P, B, N)
    om0 = jax.random.uniform(ko, (P, B, 1, 1), dtype=jnp.float32).reshape(P, B)

    big = lambda i: (0, i, 0)
    gbpq = pl.pallas_call(
        _gram_kernel,
        grid=(B // BCG,),
        in_specs=[
            pl.BlockSpec((P, BCG, N), big),
            pl.BlockSpec((P, BCG, N), big),
        ],
        out_specs=pl.BlockSpec((BCG, P, Q2), lambda i: (i, 0, 0)),
        out_shape=jax.ShapeDtypeStruct((B, P, Q2), jnp.float32),
    )(x, a0)
    g2 = jnp.transpose(gbpq, (1, 2, 0)).reshape(P, QB)

    full = lambda shape: pl.BlockSpec(shape, lambda i: (0,) * len(shape))
    wo, omo = pl.pallas_call(
        _iter_kernel,
        grid=(1,),
        in_specs=[
            full((P, DEG)),
            full((P, 1)),
            full((KTOT, P, 6)),
            full((P, QB)),
            full((P, B)),
            full((P, B)),
        ],
        out_specs=[
            full((P, QB)),
            full((P, B)),
        ],
        out_shape=[
            jax.ShapeDtypeStruct((P, QB), jnp.float32),
            jax.ShapeDtypeStruct((P, B), jnp.float32),
        ],
    )(nbr, cid, hs, g2, om0, lab)

    wt = jnp.transpose(wo.reshape(P, Q2, B), (2, 0, 1))
    a_pbn = pl.pallas_call(
        _recon_kernel,
        grid=(B // BCR,),
        in_specs=[
            pl.BlockSpec((BCR, P, Q2), lambda i: (i, 0, 0)),
            pl.BlockSpec((P, BCR, N), big),
            pl.BlockSpec((P, BCR, N), big),
        ],
        out_specs=pl.BlockSpec((P, BCR, N), big),
        out_shape=jax.ShapeDtypeStruct((P, B, N), jnp.float32),
    )(wt, a0, x)

    a_out = a_pbn.reshape(P, B, N, 1)
    om_out = omo.reshape(P, B, 1, 1)
    return a_out, om_out
